# baseline (device time: 220212 ns/iter reference)
import jax
import jax.numpy as jnp
from jax import lax
from jax.experimental import pallas as pl
from jax.experimental.pallas import tpu as pltpu

N_DEV = 8
M = 2048
N = 2048
H_SHARD = 4096
H_CHUNK = 512
CHUNK_ROWS = M // N_DEV
N_Q = 8
Q_COLS = N // N_Q
N_STEPS = N_DEV - 1


def _mlp_body(x_ref, w1_ref, w2_ref, out_ref):
    k = pl.program_id(0)
    h = jnp.dot(x_ref[...], w1_ref[...].astype(jnp.bfloat16),
                preferred_element_type=jnp.float32)
    h = jnp.maximum(h, 0.0).astype(jnp.bfloat16)
    p = jnp.dot(h, w2_ref[...].astype(jnp.bfloat16),
                preferred_element_type=jnp.float32)

    @pl.when(k == 0)
    def _():
        out_ref[...] = p

    @pl.when(k != 0)
    def _():
        out_ref[...] = out_ref[...] + p


def _ring_pos(j):
    return jnp.where(j < 4, j, 11 - j)


def _allreduce_body(p_ref, out_ref, gath_ref, recv_buf, send_buf,
                    send_sems, recv_sems):
    my = lax.axis_index("i")
    r = _ring_pos(my)
    nxt = _ring_pos(jnp.mod(r + 1, N_DEV))
    prv = _ring_pos(jnp.mod(r - 1, N_DEV))
    rr = jnp.mod(N_DEV - r, N_DEV)

    q_rd = [r] * (N_Q // 2) + [rr] * (N_Q // 2)
    q_tgt = [nxt] * (N_Q // 2) + [prv] * (N_Q // 2)

    barrier_sem = pltpu.get_barrier_semaphore()
    for nbr in (nxt, prv):
        pltpu.semaphore_signal(
            barrier_sem, inc=1,
            device_id=(nbr,), device_id_type=pltpu.DeviceIdType.MESH,
        )
    pltpu.semaphore_wait(barrier_sem, 2)

    def p_piece(c, q):
        return p_ref[pl.ds(c * CHUNK_ROWS, CHUNK_ROWS),
                     pl.ds(q * Q_COLS, Q_COLS)]

    def gath_piece(c, q):
        return gath_ref.at[pl.ds(c * CHUNK_ROWS, CHUNK_ROWS),
                           pl.ds(q * Q_COLS, Q_COLS)]

    def start_rs(q, s):
        rdma = pltpu.make_async_remote_copy(
            src_ref=send_buf.at[q],
            dst_ref=recv_buf.at[q, s],
            send_sem=send_sems.at[q, s],
            recv_sem=recv_sems.at[q, s],
            device_id=(q_tgt[q],),
            device_id_type=pltpu.DeviceIdType.MESH,
        )
        rdma.start()
        return rdma

    def start_ag(q, h):
        send_c = jnp.mod(q_rd[q] + 1 - (h - N_STEPS), N_DEV)
        rdma = pltpu.make_async_remote_copy(
            src_ref=gath_piece(send_c, q),
            dst_ref=gath_piece(send_c, q),
            send_sem=send_sems.at[q, h],
            recv_sem=recv_sems.at[q, h],
            device_id=(q_tgt[q],),
            device_id_type=pltpu.DeviceIdType.MESH,
        )
        rdma.start()
        return rdma

    inflight = []
    for q in range(N_Q):
        send_buf[q] = p_piece(jnp.mod(q_rd[q], N_DEV), q).astype(jnp.bfloat16)
        inflight.append(start_rs(q, 0))

    for h in range(2 * N_STEPS):
        for q in range(N_Q):
            inflight[q].wait()
            if h < N_STEPS - 1:
                recv_c = jnp.mod(q_rd[q] - h - 1, N_DEV)
                val = recv_buf[q, h].astype(jnp.float32) + p_piece(recv_c, q)
                send_buf[q] = val.astype(jnp.bfloat16)
                inflight[q] = start_rs(q, h + 1)
            elif h == N_STEPS - 1:
                own_c = jnp.mod(q_rd[q] + 1, N_DEV)
                val = recv_buf[q, h].astype(jnp.float32) + p_piece(own_c, q)
                gath_piece(own_c, q)[...] = val.astype(jnp.bfloat16)
                inflight[q] = start_ag(q, h + 1)
                out_ref[pl.ds(own_c * CHUNK_ROWS, CHUNK_ROWS),
                        pl.ds(q * Q_COLS, Q_COLS)] = val
            else:
                if h < 2 * N_STEPS - 1:
                    inflight[q] = start_ag(q, h + 1)
                recv_c = jnp.mod(q_rd[q] - (h - N_STEPS), N_DEV)
                out_ref[pl.ds(recv_c * CHUNK_ROWS, CHUNK_ROWS),
                        pl.ds(q * Q_COLS, Q_COLS)] = (
                    gath_ref[pl.ds(recv_c * CHUNK_ROWS, CHUNK_ROWS),
                             pl.ds(q * Q_COLS, Q_COLS)].astype(jnp.float32))


def kernel(x, W1, W2):
    xb = x.astype(jnp.bfloat16)
    partial = pl.pallas_call(
        _mlp_body,
        grid=(H_SHARD // H_CHUNK,),
        in_specs=[
            pl.BlockSpec((M, 2048), lambda k: (0, 0)),
            pl.BlockSpec((2048, H_CHUNK), lambda k: (0, k)),
            pl.BlockSpec((H_CHUNK, N), lambda k: (k, 0)),
        ],
        out_specs=pl.BlockSpec((M, N), lambda k: (0, 0)),
        out_shape=jax.ShapeDtypeStruct((M, N), jnp.float32),
        compiler_params=pltpu.CompilerParams(
            vmem_limit_bytes=60 * 1024 * 1024,
        ),
    )(xb, W1, W2)

    return pl.pallas_call(
        _allreduce_body,
        out_shape=jax.ShapeDtypeStruct((M, N), jnp.float32),
        in_specs=[pl.BlockSpec(memory_space=pltpu.VMEM)],
        out_specs=pl.BlockSpec(memory_space=pltpu.VMEM),
        scratch_shapes=[
            pltpu.VMEM((M, N), jnp.bfloat16),
            pltpu.VMEM((N_Q, N_STEPS, CHUNK_ROWS, Q_COLS), jnp.bfloat16),
            pltpu.VMEM((N_Q, CHUNK_ROWS, Q_COLS), jnp.bfloat16),
            pltpu.SemaphoreType.DMA((N_Q, 2 * N_STEPS)),
            pltpu.SemaphoreType.DMA((N_Q, 2 * N_STEPS)),
        ],
        compiler_params=pltpu.CompilerParams(
            collective_id=0,
            vmem_limit_bytes=60 * 1024 * 1024,
        ),
    )(partial)


# device time: 200861 ns/iter; 1.0963x vs baseline; 1.0963x over previous
import jax
import jax.numpy as jnp
from jax import lax
from jax.experimental import pallas as pl
from jax.experimental.pallas import tpu as pltpu

N_DEV = 8
M = 2048
N = 2048
K = 2048
H_SHARD = 4096
H_CHUNK = 512
N_Q = 8
BAND = M // N_Q
CHUNK_COLS = N // N_DEV
N_STEPS = N_DEV - 1


def _dot1_body(x_ref, w1_ref, h_ref):
    h = jnp.dot(x_ref[...], w1_ref[...].astype(jnp.bfloat16),
                preferred_element_type=jnp.float32)
    h_ref[...] = jnp.maximum(h, 0.0).astype(jnp.bfloat16)


def _ring_pos(j):
    return jnp.where(j < 4, j, 11 - j)


def _fused_body(h_ref, w2_ref, out_ref, gath_ref, recv_buf, send_buf,
                piece_buf, w2v_ref, w2_sems, send_sems, recv_sems):
    my = lax.axis_index("i")
    r = _ring_pos(my)
    nxt = _ring_pos(jnp.mod(r + 1, N_DEV))
    prv = _ring_pos(jnp.mod(r - 1, N_DEV))
    rr = jnp.mod(N_DEV - r, N_DEV)

    q_rd = [r] * (N_Q // 2) + [rr] * (N_Q // 2)
    q_tgt = [nxt] * (N_Q // 2) + [prv] * (N_Q // 2)
    d_rd = (r, rr)

    def w2_fetch(d, k):
        c = jnp.mod(d_rd[d] - k, N_DEV)
        cp = pltpu.make_async_copy(
            w2_ref.at[:, pl.ds(c * CHUNK_COLS, CHUNK_COLS)],
            w2v_ref.at[d, k % 2],
            w2_sems.at[d, k % 2],
        )
        cp.start()
        return cp

    w2_cps = {(d, k): w2_fetch(d, k) for d in range(2) for k in range(2)}

    barrier_sem = pltpu.get_barrier_semaphore()
    for nbr in (nxt, prv):
        pltpu.semaphore_signal(
            barrier_sem, inc=1,
            device_id=(nbr,), device_id_type=pltpu.DeviceIdType.MESH,
        )
    pltpu.semaphore_wait(barrier_sem, 2)

    def piece(q, k):
        d = 0 if q < N_Q // 2 else 1
        return jnp.dot(h_ref[pl.ds(q * BAND, BAND), :],
                       w2v_ref[d, k % 2],
                       preferred_element_type=jnp.float32)

    def out_piece(c, q):
        return out_ref.at[pl.ds(q * BAND, BAND),
                          pl.ds(c * CHUNK_COLS, CHUNK_COLS)]

    def gath_piece(c, q):
        return gath_ref.at[pl.ds(q * BAND, BAND),
                           pl.ds(c * CHUNK_COLS, CHUNK_COLS)]

    def start_rs(q, s):
        rdma = pltpu.make_async_remote_copy(
            src_ref=send_buf.at[q],
            dst_ref=recv_buf.at[q, s],
            send_sem=send_sems.at[q, s],
            recv_sem=recv_sems.at[q, s],
            device_id=(q_tgt[q],),
            device_id_type=pltpu.DeviceIdType.MESH,
        )
        rdma.start()
        return rdma

    def start_ag(q, h):
        send_c = jnp.mod(q_rd[q] + 1 - (h - N_STEPS), N_DEV)
        rdma = pltpu.make_async_remote_copy(
            src_ref=gath_piece(send_c, q),
            dst_ref=gath_piece(send_c, q),
            send_sem=send_sems.at[q, h],
            recv_sem=recv_sems.at[q, h],
            device_id=(q_tgt[q],),
            device_id_type=pltpu.DeviceIdType.MESH,
        )
        rdma.start()
        return rdma

    inflight = []
    for d in range(2):
        w2_cps[(d, 0)].wait()
    for q in range(N_Q):
        send_buf[q] = piece(q, 0).astype(jnp.bfloat16)
        inflight.append(start_rs(q, 0))

    for s in range(N_STEPS):
        if s + 2 < N_DEV:
            for d in range(2):
                w2_cps[(d, s + 2)] = w2_fetch(d, s + 2)
        for d in range(2):
            w2_cps[(d, s + 1)].wait()
        for q in range(N_Q):
            piece_buf[q] = piece(q, s + 1)
        for q in range(N_Q):
            inflight[q].wait()
            val = recv_buf[q, s].astype(jnp.float32) + piece_buf[q]
            if s < N_STEPS - 1:
                send_buf[q] = val.astype(jnp.bfloat16)
                inflight[q] = start_rs(q, s + 1)
            else:
                own_c = jnp.mod(q_rd[q] + 1, N_DEV)
                gath_piece(own_c, q)[...] = val.astype(jnp.bfloat16)
                inflight[q] = start_ag(q, s + 1)
                out_piece(own_c, q)[...] = val

    for h in range(N_STEPS, 2 * N_STEPS):
        for q in range(N_Q):
            inflight[q].wait()
            if h < 2 * N_STEPS - 1:
                inflight[q] = start_ag(q, h + 1)
            recv_c = jnp.mod(q_rd[q] - (h - N_STEPS), N_DEV)
            out_piece(recv_c, q)[...] = (
                gath_ref[pl.ds(q * BAND, BAND),
                         pl.ds(recv_c * CHUNK_COLS, CHUNK_COLS)]
                .astype(jnp.float32))


def kernel(x, W1, W2):
    xb = x.astype(jnp.bfloat16)
    w2b = W2.astype(jnp.bfloat16)

    h = pl.pallas_call(
        _dot1_body,
        grid=(H_SHARD // H_CHUNK,),
        in_specs=[
            pl.BlockSpec((M, K), lambda k: (0, 0)),
            pl.BlockSpec((K, H_CHUNK), lambda k: (0, k)),
        ],
        out_specs=pl.BlockSpec((M, H_CHUNK), lambda k: (0, k)),
        out_shape=jax.ShapeDtypeStruct((M, H_SHARD), jnp.bfloat16),
        compiler_params=pltpu.CompilerParams(
            vmem_limit_bytes=60 * 1024 * 1024,
        ),
    )(xb, W1)

    return pl.pallas_call(
        _fused_body,
        out_shape=jax.ShapeDtypeStruct((M, N), jnp.float32),
        in_specs=[
            pl.BlockSpec(memory_space=pltpu.VMEM),
            pl.BlockSpec(memory_space=pltpu.MemorySpace.HBM),
        ],
        out_specs=pl.BlockSpec(memory_space=pltpu.VMEM),
        scratch_shapes=[
            pltpu.VMEM((M, N), jnp.bfloat16),
            pltpu.VMEM((N_Q, N_STEPS, BAND, CHUNK_COLS), jnp.bfloat16),
            pltpu.VMEM((N_Q, BAND, CHUNK_COLS), jnp.bfloat16),
            pltpu.VMEM((N_Q, BAND, CHUNK_COLS), jnp.float32),
            pltpu.VMEM((2, 2, H_SHARD, CHUNK_COLS), jnp.bfloat16),
            pltpu.SemaphoreType.DMA((2, 2)),
            pltpu.SemaphoreType.DMA((N_Q, 2 * N_STEPS)),
            pltpu.SemaphoreType.DMA((N_Q, 2 * N_STEPS)),
        ],
        compiler_params=pltpu.CompilerParams(
            collective_id=0,
            vmem_limit_bytes=62 * 1024 * 1024,
        ),
    )(h, w2b)


# device time: 194755 ns/iter; 1.1307x vs baseline; 1.0314x over previous
import jax
import jax.numpy as jnp
from jax import lax
from jax.experimental import pallas as pl
from jax.experimental.pallas import tpu as pltpu

N_DEV = 8
M = 2048
N = 2048
K = 2048
H_SHARD = 4096
H_CHUNK = 512
N_Q = 8
BAND = M // N_Q
CHUNK_COLS = N // N_DEV
N_STEPS = N_DEV - 1


def _dot1_body(x_ref, w1_ref, h_ref, xb_ref):
    @pl.when(pl.program_id(0) == 0)
    def _():
        xb_ref[...] = x_ref[...].astype(jnp.bfloat16)

    h = jnp.dot(xb_ref[...], w1_ref[...].astype(jnp.bfloat16),
                preferred_element_type=jnp.float32)
    h_ref[...] = jnp.maximum(h, 0.0).astype(jnp.bfloat16)


def _ring_pos(j):
    return jnp.where(j < 4, j, 11 - j)


def _fused_body(h_ref, w2_ref, out_ref, gath_ref, recv_buf, send_buf,
                piece_buf, w2v_ref, w2_sems, send_sems, recv_sems):
    my = lax.axis_index("i")
    r = _ring_pos(my)
    nxt = _ring_pos(jnp.mod(r + 1, N_DEV))
    prv = _ring_pos(jnp.mod(r - 1, N_DEV))
    rr = jnp.mod(N_DEV - r, N_DEV)

    q_rd = [r] * (N_Q // 2) + [rr] * (N_Q // 2)
    q_tgt = [nxt] * (N_Q // 2) + [prv] * (N_Q // 2)
    d_rd = (r, rr)

    def w2_fetch(d, k):
        c = jnp.mod(d_rd[d] - k, N_DEV)
        cp = pltpu.make_async_copy(
            w2_ref.at[:, pl.ds(c * CHUNK_COLS, CHUNK_COLS)],
            w2v_ref.at[d, k % 2],
            w2_sems.at[d, k % 2],
        )
        cp.start()
        return cp

    w2_cps = {(d, k): w2_fetch(d, k) for d in range(2) for k in range(2)}

    barrier_sem = pltpu.get_barrier_semaphore()
    for nbr in (nxt, prv):
        pltpu.semaphore_signal(
            barrier_sem, inc=1,
            device_id=(nbr,), device_id_type=pltpu.DeviceIdType.MESH,
        )
    pltpu.semaphore_wait(barrier_sem, 2)

    def piece(q, k):
        d = 0 if q < N_Q // 2 else 1
        return jnp.dot(h_ref[pl.ds(q * BAND, BAND), :],
                       w2v_ref[d, k % 2],
                       preferred_element_type=jnp.float32)

    def out_piece(c, q):
        return out_ref.at[pl.ds(q * BAND, BAND),
                          pl.ds(c * CHUNK_COLS, CHUNK_COLS)]

    def gath_piece(c, q):
        return gath_ref.at[pl.ds(q * BAND, BAND),
                           pl.ds(c * CHUNK_COLS, CHUNK_COLS)]

    def start_rs(q, s):
        rdma = pltpu.make_async_remote_copy(
            src_ref=send_buf.at[q],
            dst_ref=recv_buf.at[q, s],
            send_sem=send_sems.at[q, s],
            recv_sem=recv_sems.at[q, s],
            device_id=(q_tgt[q],),
            device_id_type=pltpu.DeviceIdType.MESH,
        )
        rdma.start()
        return rdma

    def start_ag(q, h):
        send_c = jnp.mod(q_rd[q] + 1 - (h - N_STEPS), N_DEV)
        rdma = pltpu.make_async_remote_copy(
            src_ref=gath_piece(send_c, q),
            dst_ref=gath_piece(send_c, q),
            send_sem=send_sems.at[q, h],
            recv_sem=recv_sems.at[q, h],
            device_id=(q_tgt[q],),
            device_id_type=pltpu.DeviceIdType.MESH,
        )
        rdma.start()
        return rdma

    inflight = []
    for d in range(2):
        w2_cps[(d, 0)].wait()
    for q in range(N_Q):
        send_buf[q] = piece(q, 0).astype(jnp.bfloat16)
        inflight.append(start_rs(q, 0))

    for s in range(N_STEPS):
        if s + 2 < N_DEV:
            for d in range(2):
                w2_cps[(d, s + 2)] = w2_fetch(d, s + 2)
        for d in range(2):
            w2_cps[(d, s + 1)].wait()
        for q in range(N_Q):
            piece_buf[q] = piece(q, s + 1)
        for q in range(N_Q):
            inflight[q].wait()
            val = recv_buf[q, s].astype(jnp.float32) + piece_buf[q]
            if s < N_STEPS - 1:
                send_buf[q] = val.astype(jnp.bfloat16)
                inflight[q] = start_rs(q, s + 1)
            else:
                own_c = jnp.mod(q_rd[q] + 1, N_DEV)
                gath_piece(own_c, q)[...] = val.astype(jnp.bfloat16)
                inflight[q] = start_ag(q, s + 1)
                out_piece(own_c, q)[...] = val

    for h in range(N_STEPS, 2 * N_STEPS):
        for q in range(N_Q):
            inflight[q].wait()
            if h < 2 * N_STEPS - 1:
                inflight[q] = start_ag(q, h + 1)
            recv_c = jnp.mod(q_rd[q] - (h - N_STEPS), N_DEV)
            out_piece(recv_c, q)[...] = (
                gath_ref[pl.ds(q * BAND, BAND),
                         pl.ds(recv_c * CHUNK_COLS, CHUNK_COLS)]
                .astype(jnp.float32))


def kernel(x, W1, W2):
    w2b = W2.astype(jnp.bfloat16)

    h = pl.pallas_call(
        _dot1_body,
        grid=(H_SHARD // H_CHUNK,),
        in_specs=[
            pl.BlockSpec((M, K), lambda k: (0, 0)),
            pl.BlockSpec((K, H_CHUNK), lambda k: (0, k)),
        ],
        out_specs=pl.BlockSpec((M, H_CHUNK), lambda k: (0, k)),
        out_shape=jax.ShapeDtypeStruct((M, H_SHARD), jnp.bfloat16),
        scratch_shapes=[pltpu.VMEM((M, K), jnp.bfloat16)],
        compiler_params=pltpu.CompilerParams(
            vmem_limit_bytes=60 * 1024 * 1024,
        ),
    )(x, W1)

    return pl.pallas_call(
        _fused_body,
        out_shape=jax.ShapeDtypeStruct((M, N), jnp.float32),
        in_specs=[
            pl.BlockSpec(memory_space=pltpu.VMEM),
            pl.BlockSpec(memory_space=pltpu.MemorySpace.HBM),
        ],
        out_specs=pl.BlockSpec(memory_space=pltpu.VMEM),
        scratch_shapes=[
            pltpu.VMEM((M, N), jnp.bfloat16),
            pltpu.VMEM((N_Q, N_STEPS, BAND, CHUNK_COLS), jnp.bfloat16),
            pltpu.VMEM((N_Q, BAND, CHUNK_COLS), jnp.bfloat16),
            pltpu.VMEM((N_Q, BAND, CHUNK_COLS), jnp.float32),
            pltpu.VMEM((2, 2, H_SHARD, CHUNK_COLS), jnp.bfloat16),
            pltpu.SemaphoreType.DMA((2, 2)),
            pltpu.SemaphoreType.DMA((N_Q, 2 * N_STEPS)),
            pltpu.SemaphoreType.DMA((N_Q, 2 * N_STEPS)),
        ],
        compiler_params=pltpu.CompilerParams(
            collective_id=0,
            vmem_limit_bytes=62 * 1024 * 1024,
        ),
    )(h, w2b)


# device time: 180871 ns/iter; 1.2175x vs baseline; 1.0768x over previous
import jax
import jax.numpy as jnp
from jax import lax
from jax.experimental import pallas as pl
from jax.experimental.pallas import tpu as pltpu

N_DEV = 8
M = 2048
N = 2048
K = 2048
H_SHARD = 4096
H_CHUNK = 512
N_Q = 8
BAND = M // N_Q
CHUNK_COLS = N // N_DEV
N_STEPS = N_DEV - 1


W2_ROWS = H_SHARD


def _dot1_body(x_ref, w1_ref, w2_ref, h_ref, w2b_ref,
               xb_ref, w2stage, w2bstage, in_sems, out_sems):
    k = pl.program_id(0)
    slot = lax.rem(k, 2)

    def w2_in(kk, sl):
        return pltpu.make_async_copy(
            w2_ref.at[pl.ds(kk * H_CHUNK, H_CHUNK), :],
            w2stage.at[sl], in_sems.at[sl])

    def w2_out(kk, sl):
        return pltpu.make_async_copy(
            w2bstage.at[sl],
            w2b_ref.at[pl.ds(kk * H_CHUNK, H_CHUNK), :], out_sems.at[sl])

    @pl.when(k >= 2)
    def _():
        w2_out(k - 2, slot).wait()

    w2_in(k, slot).start()

    @pl.when(k == 0)
    def _():
        xb_ref[...] = x_ref[...].astype(jnp.bfloat16)

    h = jnp.dot(xb_ref[...], w1_ref[...].astype(jnp.bfloat16),
                preferred_element_type=jnp.float32)
    h_ref[...] = jnp.maximum(h, 0.0).astype(jnp.bfloat16)

    w2_in(k, slot).wait()
    w2bstage[slot] = w2stage[slot].astype(jnp.bfloat16)
    w2_out(k, slot).start()

    @pl.when(k == H_SHARD // H_CHUNK - 1)
    def _():
        w2_out(k - 1, lax.rem(k - 1, 2)).wait()
        w2_out(k, slot).wait()


def _ring_pos(j):
    return jnp.where(j < 4, j, 11 - j)


def _fused_body(h_ref, w2_ref, out_ref, gath_ref, recv_buf, send_buf,
                piece_buf, w2v_ref, w2_sems, send_sems, recv_sems):
    my = lax.axis_index("i")
    r = _ring_pos(my)
    nxt = _ring_pos(jnp.mod(r + 1, N_DEV))
    prv = _ring_pos(jnp.mod(r - 1, N_DEV))
    rr = jnp.mod(N_DEV - r, N_DEV)

    q_rd = [r] * (N_Q // 2) + [rr] * (N_Q // 2)
    q_tgt = [nxt] * (N_Q // 2) + [prv] * (N_Q // 2)
    d_rd = (r, rr)

    def w2_fetch(d, k):
        c = jnp.mod(d_rd[d] - k, N_DEV)
        cp = pltpu.make_async_copy(
            w2_ref.at[:, pl.ds(c * CHUNK_COLS, CHUNK_COLS)],
            w2v_ref.at[d, k % 2],
            w2_sems.at[d, k % 2],
        )
        cp.start()
        return cp

    w2_cps = {(d, k): w2_fetch(d, k) for d in range(2) for k in range(2)}

    barrier_sem = pltpu.get_barrier_semaphore()
    for nbr in (nxt, prv):
        pltpu.semaphore_signal(
            barrier_sem, inc=1,
            device_id=(nbr,), device_id_type=pltpu.DeviceIdType.MESH,
        )
    pltpu.semaphore_wait(barrier_sem, 2)

    def piece(q, k):
        d = 0 if q < N_Q // 2 else 1
        return jnp.dot(h_ref[pl.ds(q * BAND, BAND), :],
                       w2v_ref[d, k % 2],
                       preferred_element_type=jnp.float32)

    def out_piece(c, q):
        return out_ref.at[pl.ds(q * BAND, BAND),
                          pl.ds(c * CHUNK_COLS, CHUNK_COLS)]

    def gath_piece(c, q):
        return gath_ref.at[pl.ds(q * BAND, BAND),
                           pl.ds(c * CHUNK_COLS, CHUNK_COLS)]

    def start_rs(q, s):
        rdma = pltpu.make_async_remote_copy(
            src_ref=send_buf.at[q],
            dst_ref=recv_buf.at[q, s],
            send_sem=send_sems.at[q, s],
            recv_sem=recv_sems.at[q, s],
            device_id=(q_tgt[q],),
            device_id_type=pltpu.DeviceIdType.MESH,
        )
        rdma.start()
        return rdma

    def start_ag(q, h):
        send_c = jnp.mod(q_rd[q] + 1 - (h - N_STEPS), N_DEV)
        rdma = pltpu.make_async_remote_copy(
            src_ref=gath_piece(send_c, q),
            dst_ref=gath_piece(send_c, q),
            send_sem=send_sems.at[q, h],
            recv_sem=recv_sems.at[q, h],
            device_id=(q_tgt[q],),
            device_id_type=pltpu.DeviceIdType.MESH,
        )
        rdma.start()
        return rdma

    inflight = []
    for d in range(2):
        w2_cps[(d, 0)].wait()
    for q in range(N_Q):
        send_buf[q] = piece(q, 0).astype(jnp.bfloat16)
        inflight.append(start_rs(q, 0))

    for s in range(N_STEPS):
        if s + 2 < N_DEV:
            for d in range(2):
                w2_cps[(d, s + 2)] = w2_fetch(d, s + 2)
        for d in range(2):
            w2_cps[(d, s + 1)].wait()
        for q in range(N_Q):
            piece_buf[q] = piece(q, s + 1)
        for q in range(N_Q):
            inflight[q].wait()
            val = recv_buf[q, s].astype(jnp.float32) + piece_buf[q]
            if s < N_STEPS - 1:
                send_buf[q] = val.astype(jnp.bfloat16)
                inflight[q] = start_rs(q, s + 1)
            else:
                own_c = jnp.mod(q_rd[q] + 1, N_DEV)
                gath_piece(own_c, q)[...] = val.astype(jnp.bfloat16)
                inflight[q] = start_ag(q, s + 1)
                out_piece(own_c, q)[...] = val

    for h in range(N_STEPS, 2 * N_STEPS):
        for q in range(N_Q):
            inflight[q].wait()
            if h < 2 * N_STEPS - 1:
                inflight[q] = start_ag(q, h + 1)
            recv_c = jnp.mod(q_rd[q] - (h - N_STEPS), N_DEV)
            out_piece(recv_c, q)[...] = (
                gath_ref[pl.ds(q * BAND, BAND),
                         pl.ds(recv_c * CHUNK_COLS, CHUNK_COLS)]
                .astype(jnp.float32))


def kernel(x, W1, W2):
    h, w2b = pl.pallas_call(
        _dot1_body,
        grid=(H_SHARD // H_CHUNK,),
        in_specs=[
            pl.BlockSpec((M, K), lambda k: (0, 0)),
            pl.BlockSpec((K, H_CHUNK), lambda k: (0, k)),
            pl.BlockSpec(memory_space=pltpu.MemorySpace.HBM),
        ],
        out_specs=[
            pl.BlockSpec((M, H_CHUNK), lambda k: (0, k)),
            pl.BlockSpec(memory_space=pltpu.MemorySpace.HBM),
        ],
        out_shape=[
            jax.ShapeDtypeStruct((M, H_SHARD), jnp.bfloat16),
            jax.ShapeDtypeStruct((W2_ROWS, N), jnp.bfloat16),
        ],
        scratch_shapes=[
            pltpu.VMEM((M, K), jnp.bfloat16),
            pltpu.VMEM((2, H_CHUNK, N), jnp.float32),
            pltpu.VMEM((2, H_CHUNK, N), jnp.bfloat16),
            pltpu.SemaphoreType.DMA((2,)),
            pltpu.SemaphoreType.DMA((2,)),
        ],
        compiler_params=pltpu.CompilerParams(
            vmem_limit_bytes=60 * 1024 * 1024,
        ),
    )(x, W1, W2)

    return pl.pallas_call(
        _fused_body,
        out_shape=jax.ShapeDtypeStruct((M, N), jnp.float32),
        in_specs=[
            pl.BlockSpec(memory_space=pltpu.VMEM),
            pl.BlockSpec(memory_space=pltpu.MemorySpace.HBM),
        ],
        out_specs=pl.BlockSpec(memory_space=pltpu.VMEM),
        scratch_shapes=[
            pltpu.VMEM((M, N), jnp.bfloat16),
            pltpu.VMEM((N_Q, N_STEPS, BAND, CHUNK_COLS), jnp.bfloat16),
            pltpu.VMEM((N_Q, BAND, CHUNK_COLS), jnp.bfloat16),
            pltpu.VMEM((N_Q, BAND, CHUNK_COLS), jnp.float32),
            pltpu.VMEM((2, 2, H_SHARD, CHUNK_COLS), jnp.bfloat16),
            pltpu.SemaphoreType.DMA((2, 2)),
            pltpu.SemaphoreType.DMA((N_Q, 2 * N_STEPS)),
            pltpu.SemaphoreType.DMA((N_Q, 2 * N_STEPS)),
        ],
        compiler_params=pltpu.CompilerParams(
            collective_id=0,
            vmem_limit_bytes=62 * 1024 * 1024,
        ),
    )(h, w2b)
